# Initial kernel scaffold; baseline (speedup 1.0000x reference)
import jax
import jax.numpy as jnp
from jax.experimental import pallas as pl


def kernel(feature):
    raise NotImplementedError("write your pallas kernel here")


# speedup vs baseline: 1.3798x; 1.3798x over previous
import jax
import jax.numpy as jnp
from jax.experimental import pallas as pl

_MAX_SEQ_LEN = 2048
_BLOCKS = 8
_OUT_ROWS = _MAX_SEQ_LEN // _BLOCKS


def _subsample_kernel(c_ref, in_ref, out_ref):
    c = c_ref[0, :, :]
    acc = jnp.zeros(out_ref.shape, dtype=out_ref.dtype)
    for j in range(8):
        mask = c == j
        acc = acc + jnp.where(mask, in_ref[:, j, :], 0.0)
    out_ref[...] = acc


def kernel(feature):
    T, D = feature.shape
    r = jnp.linspace(0.0, float(T - 1), _MAX_SEQ_LEN).astype(jnp.int32)
    c = (r - 8 * jnp.arange(_MAX_SEQ_LEN, dtype=jnp.int32)).reshape(
        _BLOCKS, _OUT_ROWS, 1
    )
    feat3 = feature.reshape(_MAX_SEQ_LEN, _BLOCKS, D)
    return pl.pallas_call(
        _subsample_kernel,
        grid=(_BLOCKS,),
        in_specs=[
            pl.BlockSpec((1, _OUT_ROWS, 1), lambda k: (k, 0, 0)),
            pl.BlockSpec((_OUT_ROWS, _BLOCKS, D), lambda k: (k, 0, 0)),
        ],
        out_specs=pl.BlockSpec((_OUT_ROWS, D), lambda k: (k, 0)),
        out_shape=jax.ShapeDtypeStruct((_MAX_SEQ_LEN, D), feature.dtype),
    )(c, feat3)
